# baseline (speedup 1.0000x reference)
import functools

import jax
import jax.numpy as jnp
from jax import lax
from jax.experimental import pallas as pl
from jax.experimental.pallas import tpu as pltpu
from jax.experimental.pallas import tpu_sc as plsc

POOL = 7
SCALE = 0.0625
H = 50
W = 50
C = 256
NROIS = 1000
NSZ = 9
NTBL = NSZ * NSZ
ZROW = NTBL * H * W
TROWS = (NTBL + 1) * H * W
RPAD = 1024
IDXW = 56
BINS = POOL * POOL


def _table_kernel(padf_ref, out_ref, a_ref, b_ref):
    t = pl.program_id(0)
    ww = t % NSZ
    P = H + 8

    @pl.when(t == 0)
    def _():
        a_ref[...] = padf_ref[...]

    @pl.when((ww == 0) & (t > 0))
    def _():
        a = a_ref[...]
        sh1 = jnp.concatenate([a[1:], a[P - 1:]], axis=0)
        a_ref[...] = jnp.maximum(a, sh1)

    b = b_ref[...]
    bs1 = jnp.concatenate([b[:, 1:, :], b[:, P - 1:, :]], axis=1)
    b_new = jnp.where(ww == 0, a_ref[0:H], jnp.maximum(b, bs1))
    b_ref[...] = b_new
    out_ref[...] = jnp.where(
        t >= NTBL, jnp.float32(0.0), b_new[:, 0:W, :]).reshape(out_ref.shape)


def _build_table(padf):
    t3 = pl.pallas_call(
        _table_kernel,
        grid=(NTBL + 1,),
        in_specs=[pl.BlockSpec((H + 8, W + 8, C), lambda t: (0, 0, 0))],
        out_specs=pl.BlockSpec((1, H * W, C), lambda t: (t, 0, 0)),
        out_shape=jax.ShapeDtypeStruct((NTBL + 1, H * W, C), jnp.float32),
        scratch_shapes=[
            pltpu.VMEM((H + 8, W + 8, C), jnp.float32),
            pltpu.VMEM((H, W + 8, C), jnp.float32),
        ],
    )(padf)
    return t3.reshape(TROWS, C)


def _idx_kernel(rois_ref, idx_ref):
    x1 = rois_ref[1:2, :]
    y1 = rois_ref[2:3, :]
    x2 = rois_ref[3:4, :]
    y2 = rois_ref[4:5, :]

    def bounds(lo, hi, size):
        start = jnp.round(lo * SCALE).astype(jnp.int32)
        end = jnp.round(hi * SCALE).astype(jnp.int32)
        length = jnp.maximum(end - start + 1, 1).astype(jnp.float32)
        binsz = length / float(POOL)
        p = lax.broadcasted_iota(jnp.int32, (POOL, 1), 0).astype(jnp.float32)
        bstart = jnp.clip(jnp.floor(p * binsz).astype(jnp.int32) + start, 0, size)
        bend = jnp.clip(jnp.ceil((p + 1.0) * binsz).astype(jnp.int32) + start, 0, size)
        return bstart, bend - bstart

    hs, szh = bounds(y1, y2, H)
    ws, szw = bounds(x1, x2, W)
    tbl = (szh[:, None, :] - 1) * NSZ + (szw[None, :, :] - 1)
    row = tbl * (H * W) + hs[:, None, :] * W + ws[None, :, :]
    empty = (szh[:, None, :] <= 0) | (szw[None, :, :] <= 0)
    q = jnp.where(empty, ZROW, row).reshape(BINS, NROIS)
    pad = jnp.full((IDXW - BINS, NROIS), ZROW, jnp.int32)
    idx_ref[...] = jnp.concatenate([q, pad], axis=0)


def _build_idx(rois8):
    return pl.pallas_call(
        _idx_kernel,
        out_shape=jax.ShapeDtypeStruct((IDXW, NROIS), jnp.int32),
    )(rois8)


_NC = 2
_NS = 16
_NW = _NC * _NS
_RPT = RPAD // _NW
_PAIRS = _RPT // 2


@functools.cache
def _make_sc_pool():
    @functools.partial(
        pl.kernel,
        mesh=plsc.VectorSubcoreMesh(core_axis_name="c", subcore_axis_name="s"),
        compiler_params=pltpu.CompilerParams(needs_layout_passes=False),
        out_type=jax.ShapeDtypeStruct((RPAD // 2, 2 * C * BINS), jnp.float32),
        scratch_types=[
            pltpu.VMEM((_PAIRS, 2 * IDXW), jnp.int32),
            pltpu.VMEM((2, 2 * IDXW, C), jnp.float32),
            pltpu.VMEM((2 * 2 * C * BINS,), jnp.float32),
            pltpu.SemaphoreType.DMA((2,)),
            pltpu.SemaphoreType.DMA((2,)),
        ],
    )
    def _sc_pool(table_hbm, idx_hbm, out_hbm, idx_all, rows2, out2,
                 sem_g, sem_o):
        wid = lax.axis_index("s") * _NC + lax.axis_index("c")
        pltpu.sync_copy(
            idx_hbm.at[pl.ds(wid * _PAIRS, _PAIRS)], idx_all)
        lane49 = lax.broadcasted_iota(jnp.int32, (16,), 0) * BINS
        OSZ = 2 * C * BINS

        def gather_desc(p):
            b = p & 1
            return pltpu.make_async_copy(
                table_hbm.at[idx_all.at[p]], rows2.at[b], sem_g.at[b])

        def out_desc(p):
            b = p & 1
            return pltpu.make_async_copy(
                out2.at[pl.ds(b * OSZ, OSZ)],
                out_hbm.at[wid * _PAIRS + p], sem_o.at[b])

        gather_desc(0).start()

        def pair_body(p, carry):
            @pl.when(p + 1 < _PAIRS)
            def _():
                gather_desc(p + 1).start()

            gather_desc(p).wait()
            b = p & 1

            @pl.when(p >= 2)
            def _():
                out_desc(p - 2).wait()

            obase = b * OSZ

            def bin_body(j, c2):
                for c in range(C // 16):
                    sl = pl.ds(c * 16, 16)
                    v0 = rows2[b, j, sl]
                    plsc.store_scatter(
                        out2, [obase + lane49 + (c * 16 * BINS) + j], v0)
                    v1 = rows2[b, IDXW + j, sl]
                    plsc.store_scatter(
                        out2,
                        [obase + lane49 + (C * BINS + c * 16 * BINS) + j], v1)
                return c2

            lax.fori_loop(0, BINS, bin_body, 0)
            out_desc(p).start()
            return carry

        lax.fori_loop(0, _PAIRS, pair_body, 0)
        out_desc(_PAIRS - 2).wait()
        out_desc(_PAIRS - 1).wait()

    return _sc_pool


def kernel(feat, rois):
    feat_t = jnp.transpose(feat[0], (1, 2, 0))
    padf = jnp.pad(feat_t, ((0, 8), (0, 8), (0, 0)), mode="edge")
    rois_t = jnp.transpose(rois)
    rois8 = jnp.concatenate(
        [rois_t, jnp.zeros((3, NROIS), jnp.float32)], axis=0)
    table = _build_table(padf)
    idx_t = _build_idx(rois8)
    idx = jnp.transpose(idx_t)
    idx = jnp.concatenate(
        [idx, jnp.full((RPAD - NROIS, IDXW), ZROW, jnp.int32)], axis=0)
    idx = idx.reshape(RPAD // 2, 2 * IDXW)
    out = _make_sc_pool()(table, idx)
    out = out.reshape(RPAD, C * BINS)[:NROIS]
    return out.reshape(NROIS, C, POOL, POOL)


# speedup vs baseline: 1.0167x; 1.0167x over previous
import functools

import jax
import jax.numpy as jnp
from jax import lax
from jax.experimental import pallas as pl
from jax.experimental.pallas import tpu as pltpu
from jax.experimental.pallas import tpu_sc as plsc

POOL = 7
SCALE = 0.0625
H = 50
W = 50
C = 256
NROIS = 1000
NSZ = 9
NTBL = NSZ * NSZ
ZROW = NTBL * H * W
TROWS = (NTBL + 1) * H * W
RPAD = 1024
IDXW = 56
BINS = POOL * POOL


def _table_kernel(padf_ref, out_ref, a_ref, b_ref):
    t = pl.program_id(0)
    ww = t % NSZ
    P = H + 8

    @pl.when(t == 0)
    def _():
        a_ref[...] = padf_ref[...]

    @pl.when((ww == 0) & (t > 0))
    def _():
        a = a_ref[...]
        sh1 = jnp.concatenate([a[1:], a[P - 1:]], axis=0)
        a_ref[...] = jnp.maximum(a, sh1)

    b = b_ref[...]
    bs1 = jnp.concatenate([b[:, 1:, :], b[:, P - 1:, :]], axis=1)
    b_new = jnp.where(ww == 0, a_ref[0:H], jnp.maximum(b, bs1))
    b_ref[...] = b_new
    out_ref[...] = jnp.where(
        t >= NTBL, jnp.float32(0.0), b_new[:, 0:W, :]).reshape(out_ref.shape)


def _build_table(padf):
    t3 = pl.pallas_call(
        _table_kernel,
        grid=(NTBL + 1,),
        in_specs=[pl.BlockSpec((H + 8, W + 8, C), lambda t: (0, 0, 0))],
        out_specs=pl.BlockSpec((1, H * W, C), lambda t: (t, 0, 0)),
        out_shape=jax.ShapeDtypeStruct((NTBL + 1, H * W, C), jnp.float32),
        scratch_shapes=[
            pltpu.VMEM((H + 8, W + 8, C), jnp.float32),
            pltpu.VMEM((H, W + 8, C), jnp.float32),
        ],
    )(padf)
    return t3.reshape(TROWS, C)


def _idx_kernel(rois_ref, idx_ref):
    x1 = rois_ref[1:2, :]
    y1 = rois_ref[2:3, :]
    x2 = rois_ref[3:4, :]
    y2 = rois_ref[4:5, :]

    def bounds(lo, hi, size):
        start = jnp.round(lo * SCALE).astype(jnp.int32)
        end = jnp.round(hi * SCALE).astype(jnp.int32)
        length = jnp.maximum(end - start + 1, 1).astype(jnp.float32)
        binsz = length / float(POOL)
        p = lax.broadcasted_iota(jnp.int32, (POOL, 1), 0).astype(jnp.float32)
        bstart = jnp.clip(jnp.floor(p * binsz).astype(jnp.int32) + start, 0, size)
        bend = jnp.clip(jnp.ceil((p + 1.0) * binsz).astype(jnp.int32) + start, 0, size)
        return bstart, bend - bstart

    hs, szh = bounds(y1, y2, H)
    ws, szw = bounds(x1, x2, W)
    tbl = (szh[:, None, :] - 1) * NSZ + (szw[None, :, :] - 1)
    row = tbl * (H * W) + hs[:, None, :] * W + ws[None, :, :]
    empty = (szh[:, None, :] <= 0) | (szw[None, :, :] <= 0)
    q = jnp.where(empty, ZROW, row).reshape(BINS, NROIS)
    pad = jnp.full((IDXW - BINS, NROIS), ZROW, jnp.int32)
    idx_ref[...] = jnp.concatenate([q, pad], axis=0)


def _build_idx(rois8):
    return pl.pallas_call(
        _idx_kernel,
        out_shape=jax.ShapeDtypeStruct((IDXW, NROIS), jnp.int32),
    )(rois8)


_NC = 2
_NS = 16
_NW = _NC * _NS
_RPT = RPAD // _NW
_PAIRS = _RPT // 2


@functools.cache
def _make_sc_pool():
    @functools.partial(
        pl.kernel,
        mesh=plsc.VectorSubcoreMesh(core_axis_name="c", subcore_axis_name="s"),
        compiler_params=pltpu.CompilerParams(needs_layout_passes=False),
        out_type=jax.ShapeDtypeStruct((NROIS // 2, 2 * C * BINS), jnp.float32),
        scratch_types=[
            pltpu.VMEM((_PAIRS, 2 * IDXW), jnp.int32),
            pltpu.VMEM((2, 2 * IDXW, C), jnp.float32),
            pltpu.VMEM((2 * 2 * C * BINS,), jnp.float32),
            pltpu.SemaphoreType.DMA((2,)),
            pltpu.SemaphoreType.DMA((2,)),
        ],
    )
    def _sc_pool(table_hbm, idx_hbm, out_hbm, idx_all, rows2, out2,
                 sem_g, sem_o):
        wid = lax.axis_index("s") * _NC + lax.axis_index("c")
        pltpu.sync_copy(
            idx_hbm.at[pl.ds(wid * _PAIRS, _PAIRS)], idx_all)
        lane49 = lax.broadcasted_iota(jnp.int32, (16,), 0) * BINS
        OSZ = 2 * C * BINS

        def gather_desc(p):
            b = p & 1
            return pltpu.make_async_copy(
                table_hbm.at[idx_all.at[p]], rows2.at[b], sem_g.at[b])

        def out_desc(p):
            b = p & 1
            return pltpu.make_async_copy(
                out2.at[pl.ds(b * OSZ, OSZ)],
                out_hbm.at[wid * _PAIRS + p], sem_o.at[b])

        gather_desc(0).start()

        def pair_body(p, carry):
            @pl.when(p + 1 < _PAIRS)
            def _():
                gather_desc(p + 1).start()

            gather_desc(p).wait()
            b = p & 1

            @pl.when((p >= 2) & (wid * _PAIRS + p - 2 < NROIS // 2))
            def _():
                out_desc(p - 2).wait()

            obase = b * OSZ

            def bin_body(j, c2):
                for c in range(C // 16):
                    sl = pl.ds(c * 16, 16)
                    v0 = rows2[b, j, sl]
                    plsc.store_scatter(
                        out2, [obase + lane49 + (c * 16 * BINS) + j], v0)
                    v1 = rows2[b, IDXW + j, sl]
                    plsc.store_scatter(
                        out2,
                        [obase + lane49 + (C * BINS + c * 16 * BINS) + j], v1)
                return c2

            lax.fori_loop(0, BINS, bin_body, 0)

            @pl.when(wid * _PAIRS + p < NROIS // 2)
            def _():
                out_desc(p).start()

            return carry

        lax.fori_loop(0, _PAIRS, pair_body, 0)

        @pl.when(wid * _PAIRS + _PAIRS - 2 < NROIS // 2)
        def _():
            out_desc(_PAIRS - 2).wait()

        @pl.when(wid * _PAIRS + _PAIRS - 1 < NROIS // 2)
        def _():
            out_desc(_PAIRS - 1).wait()

    return _sc_pool


def kernel(feat, rois):
    feat_t = jnp.transpose(feat[0], (1, 2, 0))
    padf = jnp.pad(feat_t, ((0, 8), (0, 8), (0, 0)), mode="edge")
    rois_t = jnp.transpose(rois)
    rois8 = jnp.concatenate(
        [rois_t, jnp.zeros((3, NROIS), jnp.float32)], axis=0)
    table = _build_table(padf)
    idx_t = _build_idx(rois8)
    idx = jnp.transpose(idx_t)
    idx = jnp.concatenate(
        [idx, jnp.full((RPAD - NROIS, IDXW), ZROW, jnp.int32)], axis=0)
    idx = idx.reshape(RPAD // 2, 2 * IDXW)
    out = _make_sc_pool()(table, idx)
    return out.reshape(NROIS, C, POOL, POOL)


# speedup vs baseline: 2.0798x; 2.0456x over previous
import functools

import jax
import jax.numpy as jnp
from jax import lax
from jax.experimental import pallas as pl
from jax.experimental.pallas import tpu as pltpu
from jax.experimental.pallas import tpu_sc as plsc

POOL = 7
SCALE = 0.0625
H = 50
W = 50
C = 256
NROIS = 1000
NSZ = 9
NTBL = NSZ * NSZ
ZROW = NTBL * H * W
RPAD = 1024
BINS = POOL * POOL
GIDX = 112
SIDX = 98
TROWS2 = (NTBL + 1) * H * W * 2


def _table_kernel(padf_ref, out_ref, a_ref, b_ref):
    t = pl.program_id(0)
    P = H + 8

    def emit_one(k):
        ww = k % NSZ

        @pl.when(k == 0)
        def _():
            a_ref[...] = padf_ref[...]

        @pl.when((ww == 0) & (k > 0))
        def _():
            a = a_ref[...]
            sh1 = jnp.concatenate([a[1:], a[P - 1:]], axis=0)
            a_ref[...] = jnp.maximum(a, sh1)

        b = b_ref[...]
        bs1 = jnp.concatenate([b[:, 1:, :], b[:, P - 1:, :]], axis=1)
        b_new = jnp.where(ww == 0, a_ref[0:H], jnp.maximum(b, bs1))
        b_ref[...] = b_new
        v = jnp.where(k >= NTBL, jnp.float32(0.0), b_new[:, 0:W, :])
        return v.reshape(H * W * 2, C // 2)

    v0 = emit_one(2 * t)
    v1 = emit_one(2 * t + 1)
    out_ref[...] = jnp.concatenate([v0, v1], axis=0)


def _build_table(padf):
    return pl.pallas_call(
        _table_kernel,
        grid=((NTBL + 1) // 2,),
        in_specs=[pl.BlockSpec((H + 8, W + 8, C), lambda t: (0, 0, 0))],
        out_specs=pl.BlockSpec((H * W * 4, C // 2), lambda t: (t, 0)),
        out_shape=jax.ShapeDtypeStruct((TROWS2, C // 2), jnp.float32),
        scratch_shapes=[
            pltpu.VMEM((H + 8, W + 8, C), jnp.float32),
            pltpu.VMEM((H, W + 8, C), jnp.float32),
        ],
    )(padf)


def _idx_kernel(rois_ref, gidx_ref, sidx_ref):
    x1 = rois_ref[1:2, :]
    y1 = rois_ref[2:3, :]
    x2 = rois_ref[3:4, :]
    y2 = rois_ref[4:5, :]

    def bounds(lo, hi, size):
        start = jnp.round(lo * SCALE).astype(jnp.int32)
        end = jnp.round(hi * SCALE).astype(jnp.int32)
        length = jnp.maximum(end - start + 1, 1).astype(jnp.float32)
        binsz = length / float(POOL)
        p = lax.broadcasted_iota(jnp.int32, (POOL, 1), 0).astype(jnp.float32)
        bstart = jnp.clip(jnp.floor(p * binsz).astype(jnp.int32) + start, 0, size)
        bend = jnp.clip(jnp.ceil((p + 1.0) * binsz).astype(jnp.int32) + start, 0, size)
        return bstart, bend - bstart

    hs, szh = bounds(y1, y2, H)
    ws, szw = bounds(x1, x2, W)
    tbl = (szh[:, None, :] - 1) * NSZ + (szw[None, :, :] - 1)
    row = tbl * (H * W) + hs[:, None, :] * W + ws[None, :, :]
    empty = (szh[:, None, :] <= 0) | (szw[None, :, :] <= 0)
    q = jnp.where(empty, ZROW, row).reshape(BINS, 1, NROIS)
    ch = lax.broadcasted_iota(jnp.int32, (1, 2, 1), 1)
    g = (q * 2 + ch).reshape(SIDX, NROIS)
    gpad = jnp.full((GIDX - SIDX, NROIS), 2 * ZROW, jnp.int32)
    gidx_ref[...] = jnp.concatenate([g, gpad], axis=0)
    e = lax.broadcasted_iota(jnp.int32, (SIDX, 1), 0)
    ebase = (e // 2) * (RPAD_OUT := 2 * 8 * (NROIS // 8)) + (e % 2) * 8
    r = lax.broadcasted_iota(jnp.int32, (1, NROIS), 1)
    sidx_ref[...] = ebase + (r // 8) * 16 + r % 8


def _build_idx(rois8):
    return pl.pallas_call(
        _idx_kernel,
        out_shape=[
            jax.ShapeDtypeStruct((GIDX, NROIS), jnp.int32),
            jax.ShapeDtypeStruct((SIDX, NROIS), jnp.int32),
        ],
    )(rois8)


_NC = 2
_NS = 16
_NW = _NC * _NS
_RPT = RPAD // _NW
_NBUF = 4


@functools.cache
def _make_sc_pool():
    @functools.partial(
        pl.kernel,
        mesh=plsc.VectorSubcoreMesh(core_axis_name="c", subcore_axis_name="s"),
        compiler_params=pltpu.CompilerParams(needs_layout_passes=False),
        out_type=jax.ShapeDtypeStruct((2 * BINS * NROIS, C // 2), jnp.float32),
        scratch_types=[
            pltpu.VMEM((_RPT, GIDX), jnp.int32),
            pltpu.VMEM((_RPT, SIDX), jnp.int32),
            pltpu.VMEM((_NBUF, GIDX, C // 2), jnp.float32),
            pltpu.SemaphoreType.DMA((_NBUF,)),
            pltpu.SemaphoreType.DMA((_NBUF,)),
        ],
    )
    def _sc_pool(table_hbm, gidx_hbm, sidx_hbm, out_hbm, gidx_all, sidx_all,
                 rows, sem_g, sem_s):
        wid = lax.axis_index("s") * _NC + lax.axis_index("c")
        pltpu.sync_copy(gidx_hbm.at[pl.ds(wid * _RPT, _RPT)], gidx_all)
        pltpu.sync_copy(sidx_hbm.at[pl.ds(wid * _RPT, _RPT)], sidx_all)

        def g_desc(i):
            b = i & (_NBUF - 1)
            return pltpu.make_async_copy(
                table_hbm.at[gidx_all.at[i]], rows.at[b], sem_g.at[b])

        def s_desc(i):
            b = i & (_NBUF - 1)
            return pltpu.make_async_copy(
                rows.at[b, pl.ds(0, SIDX)], out_hbm.at[sidx_all.at[i]],
                sem_s.at[b])

        def ran(i):
            return wid * _RPT + i < NROIS

        g_desc(0).start()

        def roi_body(i, carry):
            @pl.when((i >= 3) & ran(i - 3))
            def _():
                s_desc(i - 3).wait()

            @pl.when(i + 1 < _RPT)
            def _():
                g_desc(i + 1).start()

            g_desc(i).wait()

            @pl.when(ran(i))
            def _():
                s_desc(i).start()

            return carry

        lax.fori_loop(0, _RPT, roi_body, 0)
        for i in range(_RPT - 3, _RPT):
            @pl.when(ran(i))
            def _():
                s_desc(i).wait()

    return _sc_pool


def kernel(feat, rois):
    feat_t = jnp.transpose(feat[0], (1, 2, 0))
    padf = jnp.pad(feat_t, ((0, 8), (0, 8), (0, 0)), mode="edge")
    rois_t = jnp.transpose(rois)
    rois8 = jnp.concatenate(
        [rois_t, jnp.zeros((3, NROIS), jnp.float32)], axis=0)
    table = _build_table(padf)
    gidx_t, sidx_t = _build_idx(rois8)
    gidx = jnp.concatenate(
        [jnp.transpose(gidx_t),
         jnp.full((RPAD - NROIS, GIDX), 2 * ZROW, jnp.int32)], axis=0)
    sidx = jnp.concatenate(
        [jnp.transpose(sidx_t),
         jnp.zeros((RPAD - NROIS, SIDX), jnp.int32)], axis=0)
    out2 = _make_sc_pool()(table, gidx, sidx)
    out6 = out2.reshape(BINS, NROIS // 8, 2, 8, C // 2)
    out = jnp.transpose(out6, (1, 3, 2, 4, 0))
    return out.reshape(NROIS, C, POOL, POOL)


# speedup vs baseline: 2.0843x; 1.0021x over previous
import functools

import jax
import jax.numpy as jnp
from jax import lax
from jax.experimental import pallas as pl
from jax.experimental.pallas import tpu as pltpu
from jax.experimental.pallas import tpu_sc as plsc

POOL = 7
SCALE = 0.0625
H = 50
W = 50
C = 256
NROIS = 1000
NSZ = 9
NTBL = NSZ * NSZ
ZROW = NTBL * H * W
RPAD = 1024
BINS = POOL * POOL
GIDX = 112
SIDX = 98
TROWS2 = (NTBL + 1) * H * W * 2


def _table_kernel(padf_ref, out_ref, a_ref, b_ref):
    t = pl.program_id(0)
    P = H + 8

    def emit_one(k):
        ww = k % NSZ

        @pl.when(k == 0)
        def _():
            a_ref[...] = padf_ref[...]

        @pl.when((ww == 0) & (k > 0))
        def _():
            a = a_ref[...]
            sh1 = jnp.concatenate([a[1:], a[P - 1:]], axis=0)
            a_ref[...] = jnp.maximum(a, sh1)

        b = b_ref[...]
        bs1 = jnp.concatenate([b[:, 1:, :], b[:, P - 1:, :]], axis=1)
        b_new = jnp.where(ww == 0, a_ref[0:H], jnp.maximum(b, bs1))
        b_ref[...] = b_new
        v = jnp.where(k >= NTBL, jnp.float32(0.0), b_new[:, 0:W, :])
        return v.reshape(H * W * 2, C // 2)

    v0 = emit_one(2 * t)
    v1 = emit_one(2 * t + 1)
    out_ref[...] = jnp.concatenate([v0, v1], axis=0)


def _build_table(padf):
    return pl.pallas_call(
        _table_kernel,
        grid=((NTBL + 1) // 2,),
        in_specs=[pl.BlockSpec((H + 8, W + 8, C), lambda t: (0, 0, 0))],
        out_specs=pl.BlockSpec((H * W * 4, C // 2), lambda t: (t, 0)),
        out_shape=jax.ShapeDtypeStruct((TROWS2, C // 2), jnp.float32),
        scratch_shapes=[
            pltpu.VMEM((H + 8, W + 8, C), jnp.float32),
            pltpu.VMEM((H, W + 8, C), jnp.float32),
        ],
    )(padf)


def _idx_kernel(rois_ref, gidx_ref, sidx_ref):
    x1 = rois_ref[1:2, :]
    y1 = rois_ref[2:3, :]
    x2 = rois_ref[3:4, :]
    y2 = rois_ref[4:5, :]

    def bounds(lo, hi, size):
        start = jnp.round(lo * SCALE).astype(jnp.int32)
        end = jnp.round(hi * SCALE).astype(jnp.int32)
        length = jnp.maximum(end - start + 1, 1).astype(jnp.float32)
        binsz = length / float(POOL)
        p = lax.broadcasted_iota(jnp.int32, (POOL, 1), 0).astype(jnp.float32)
        bstart = jnp.clip(jnp.floor(p * binsz).astype(jnp.int32) + start, 0, size)
        bend = jnp.clip(jnp.ceil((p + 1.0) * binsz).astype(jnp.int32) + start, 0, size)
        return bstart, bend - bstart

    hs, szh = bounds(y1, y2, H)
    ws, szw = bounds(x1, x2, W)
    tbl = (szh[:, None, :] - 1) * NSZ + (szw[None, :, :] - 1)
    row = tbl * (H * W) + hs[:, None, :] * W + ws[None, :, :]
    empty = (szh[:, None, :] <= 0) | (szw[None, :, :] <= 0)
    q = jnp.where(empty, ZROW, row).reshape(BINS, 1, NROIS)
    ch = lax.broadcasted_iota(jnp.int32, (1, 2, 1), 1)
    g = (q * 2 + ch).reshape(SIDX, NROIS)
    gpad = jnp.full((GIDX - SIDX, NROIS), 2 * ZROW, jnp.int32)
    gidx_ref[...] = jnp.concatenate([g, gpad], axis=0)
    e = lax.broadcasted_iota(jnp.int32, (SIDX, 1), 0)
    ebase = (e // 2) * (RPAD_OUT := 2 * 8 * (NROIS // 8)) + (e % 2) * 8
    r = lax.broadcasted_iota(jnp.int32, (1, NROIS), 1)
    sidx_ref[...] = ebase + (r // 8) * 16 + r % 8


def _build_idx(rois8):
    return pl.pallas_call(
        _idx_kernel,
        out_shape=[
            jax.ShapeDtypeStruct((GIDX, NROIS), jnp.int32),
            jax.ShapeDtypeStruct((SIDX, NROIS), jnp.int32),
        ],
    )(rois8)


_NC = 2
_NS = 16
_NW = _NC * _NS
_RPT = RPAD // _NW
_NBUF = 8
_AHEAD = 3


@functools.cache
def _make_sc_pool():
    @functools.partial(
        pl.kernel,
        mesh=plsc.VectorSubcoreMesh(core_axis_name="c", subcore_axis_name="s"),
        compiler_params=pltpu.CompilerParams(needs_layout_passes=False),
        out_type=jax.ShapeDtypeStruct((2 * BINS * NROIS, C // 2), jnp.float32),
        scratch_types=[
            pltpu.VMEM((_RPT, GIDX), jnp.int32),
            pltpu.VMEM((_RPT, SIDX), jnp.int32),
            pltpu.VMEM((_NBUF, GIDX, C // 2), jnp.float32),
            pltpu.SemaphoreType.DMA((_NBUF,)),
            pltpu.SemaphoreType.DMA((_NBUF,)),
        ],
    )
    def _sc_pool(table_hbm, gidx_hbm, sidx_hbm, out_hbm, gidx_all, sidx_all,
                 rows, sem_g, sem_s):
        wid = lax.axis_index("s") * _NC + lax.axis_index("c")
        pltpu.sync_copy(gidx_hbm.at[pl.ds(wid * _RPT, _RPT)], gidx_all)
        pltpu.sync_copy(sidx_hbm.at[pl.ds(wid * _RPT, _RPT)], sidx_all)

        def g_desc(i):
            b = i & (_NBUF - 1)
            return pltpu.make_async_copy(
                table_hbm.at[gidx_all.at[i]], rows.at[b], sem_g.at[b])

        def s_desc(i):
            b = i & (_NBUF - 1)
            return pltpu.make_async_copy(
                rows.at[b, pl.ds(0, SIDX)], out_hbm.at[sidx_all.at[i]],
                sem_s.at[b])

        def ran(i):
            return wid * _RPT + i < NROIS

        for i in range(_AHEAD):
            g_desc(i).start()

        def roi_body(i, carry):
            @pl.when((i + _AHEAD >= _NBUF) & ran(i + _AHEAD - _NBUF))
            def _():
                s_desc(i + _AHEAD - _NBUF).wait()

            @pl.when(i + _AHEAD < _RPT)
            def _():
                g_desc(i + _AHEAD).start()

            g_desc(i).wait()

            @pl.when(ran(i))
            def _():
                s_desc(i).start()

            return carry

        lax.fori_loop(0, _RPT, roi_body, 0)
        for i in range(_RPT - (_NBUF - _AHEAD), _RPT):
            @pl.when(ran(i))
            def _():
                s_desc(i).wait()

    return _sc_pool


def kernel(feat, rois):
    feat_t = jnp.transpose(feat[0], (1, 2, 0))
    padf = jnp.pad(feat_t, ((0, 8), (0, 8), (0, 0)), mode="edge")
    rois_t = jnp.transpose(rois)
    rois8 = jnp.concatenate(
        [rois_t, jnp.zeros((3, NROIS), jnp.float32)], axis=0)
    table = _build_table(padf)
    gidx_t, sidx_t = _build_idx(rois8)
    gidx = jnp.concatenate(
        [jnp.transpose(gidx_t),
         jnp.full((RPAD - NROIS, GIDX), 2 * ZROW, jnp.int32)], axis=0)
    sidx = jnp.concatenate(
        [jnp.transpose(sidx_t),
         jnp.zeros((RPAD - NROIS, SIDX), jnp.int32)], axis=0)
    out2 = _make_sc_pool()(table, gidx, sidx)
    out6 = out2.reshape(BINS, NROIS // 8, 2, 8, C // 2)
    out = jnp.transpose(out6, (1, 3, 2, 4, 0))
    return out.reshape(NROIS, C, POOL, POOL)


# speedup vs baseline: 4.4051x; 2.1135x over previous
import functools

import jax
import jax.numpy as jnp
from jax import lax
from jax.experimental import pallas as pl
from jax.experimental.pallas import tpu as pltpu
from jax.experimental.pallas import tpu_sc as plsc

POOL = 7
SCALE = 0.0625
H = 50
W = 50
C = 256
NROIS = 1000
NSZ = 9
NTBL = NSZ * NSZ
ZROW = NTBL * H * W
RPAD = 1024
BINS = POOL * POOL
GIDX = 112
SIDX = 98
TROWS2 = (NTBL + 1) * H * W * 2


def _table_kernel(padf_ref, out_ref, a_ref, b_ref):
    t = pl.program_id(0)
    P = H + 8

    def emit_one(k):
        ww = k % NSZ

        @pl.when(k == 0)
        def _():
            a_ref[...] = padf_ref[...]

        @pl.when((ww == 0) & (k > 0))
        def _():
            a = a_ref[...]
            sh1 = jnp.concatenate([a[1:], a[P - 1:]], axis=0)
            a_ref[...] = jnp.maximum(a, sh1)

        b = b_ref[...]
        bs1 = jnp.concatenate([b[:, 1:, :], b[:, P - 1:, :]], axis=1)
        b_new = jnp.where(ww == 0, a_ref[0:H], jnp.maximum(b, bs1))
        b_ref[...] = b_new
        v = jnp.where(k >= NTBL, jnp.float32(0.0), b_new[:, 0:W, :])
        return v.reshape(H * W * 2, C // 2)

    v0 = emit_one(2 * t)
    v1 = emit_one(2 * t + 1)
    out_ref[...] = jnp.concatenate([v0, v1], axis=0)


def _build_table(padf):
    return pl.pallas_call(
        _table_kernel,
        grid=((NTBL + 1) // 2,),
        in_specs=[pl.BlockSpec((H + 8, W + 8, C), lambda t: (0, 0, 0))],
        out_specs=pl.BlockSpec((H * W * 4, C // 2), lambda t: (t, 0)),
        out_shape=jax.ShapeDtypeStruct((TROWS2, C // 2), jnp.float32),
        scratch_shapes=[
            pltpu.VMEM((H + 8, W + 8, C), jnp.float32),
            pltpu.VMEM((H, W + 8, C), jnp.float32),
        ],
    )(padf)


def _idx_kernel(rois_ref, gidx_ref):
    x1 = rois_ref[1:2, :]
    y1 = rois_ref[2:3, :]
    x2 = rois_ref[3:4, :]
    y2 = rois_ref[4:5, :]

    def bounds(lo, hi, size):
        start = jnp.round(lo * SCALE).astype(jnp.int32)
        end = jnp.round(hi * SCALE).astype(jnp.int32)
        length = jnp.maximum(end - start + 1, 1).astype(jnp.float32)
        binsz = length / float(POOL)
        p = lax.broadcasted_iota(jnp.int32, (POOL, 1), 0).astype(jnp.float32)
        bstart = jnp.clip(jnp.floor(p * binsz).astype(jnp.int32) + start, 0, size)
        bend = jnp.clip(jnp.ceil((p + 1.0) * binsz).astype(jnp.int32) + start, 0, size)
        return bstart, bend - bstart

    hs, szh = bounds(y1, y2, H)
    ws, szw = bounds(x1, x2, W)
    tbl = (szh[:, None, :] - 1) * NSZ + (szw[None, :, :] - 1)
    row = tbl * (H * W) + hs[:, None, :] * W + ws[None, :, :]
    empty = (szh[:, None, :] <= 0) | (szw[None, :, :] <= 0)
    q = jnp.where(empty, ZROW, row).reshape(BINS, NROIS)
    qp = jnp.concatenate(
        [q, jnp.full((BINS, RPAD - NROIS), ZROW, jnp.int32)], axis=1)
    q3 = qp.reshape(BINS, RPAD // 8, 8)
    ch = lax.broadcasted_iota(jnp.int32, (1, 1, 2, 1), 2)
    g4 = q3[:, :, None, :] * 2 + ch
    g4 = jnp.transpose(g4, (1, 0, 2, 3))
    half_a = g4[:, 0:25].reshape(RPAD // 8, 400)
    half_b = jnp.concatenate(
        [g4[:, 25:BINS].reshape(RPAD // 8, 384),
         jnp.full((RPAD // 8, 16), 2 * ZROW, jnp.int32)], axis=1)
    gidx_ref[...] = jnp.stack(
        [half_a, half_b], axis=1).reshape(RPAD // 4, 400)


def _build_idx(rois8):
    return pl.pallas_call(
        _idx_kernel,
        out_shape=jax.ShapeDtypeStruct((RPAD // 4, 400), jnp.int32),
    )(rois8)


_NC = 2
_NS = 16
_NW = _NC * _NS
_RPT = RPAD // _NW
_NHALF = 8
_GCH = ((0, 128), (128, 128), (256, 128), (384, 16))


@functools.cache
def _make_sc_pool():
    @functools.partial(
        pl.kernel,
        mesh=plsc.VectorSubcoreMesh(core_axis_name="c", subcore_axis_name="s"),
        compiler_params=pltpu.CompilerParams(needs_layout_passes=False),
        out_type=jax.ShapeDtypeStruct((2 * BINS * NROIS, C // 2), jnp.float32),
        scratch_types=[
            pltpu.VMEM((2, 400), jnp.int32),
            pltpu.VMEM((2, 400, C // 2), jnp.float32),
            pltpu.SemaphoreType.DMA((2,)),
            pltpu.SemaphoreType.DMA((2,)),
        ],
    )
    def _sc_pool(table_hbm, gidx_hbm, out_hbm, idx_v, rows, sem_g, sem_s):
        wid = lax.axis_index("s") * _NC + lax.axis_index("c")

        def idx_copy(k):
            pltpu.sync_copy(gidx_hbm.at[wid * _NHALF + k], idx_v.at[k & 1])

        def g_descs(k):
            b = k & 1
            return [
                pltpu.make_async_copy(
                    table_hbm.at[idx_v.at[b, pl.ds(o, n)]],
                    rows.at[b, pl.ds(o, n)], sem_g.at[b])
                for o, n in _GCH
            ]

        def group_ok(k):
            return (wid * 4 + k // 2) * 8 < NROIS

        def scatter_issue(k):
            b = k & 1
            h = k % 2
            nb = 25 - h
            rtg = wid * 4 + k // 2
            stride = 16 * (NROIS // 8)
            base = h * 25 * stride + rtg * 16

            def sc_body(e, c):
                dest = base + (e >> 1) * stride + (e & 1) * 8
                pltpu.async_copy(
                    rows.at[b, pl.ds(e * 8, 8)],
                    out_hbm.at[pl.ds(dest, 8)], sem_s.at[b])
                return c

            lax.fori_loop(0, 2 * nb, sc_body, 0)

        def scatter_wait(k):
            b = k & 1
            nb = 25 - (k % 2)

            def w_body(e, c):
                pltpu.make_async_copy(
                    rows.at[b, pl.ds(0, 8)],
                    out_hbm.at[pl.ds(0, 8)], sem_s.at[b]).wait()
                return c

            lax.fori_loop(0, 2 * nb, w_body, 0)

        idx_copy(0)
        for d in g_descs(0):
            d.start()
        for k in range(_NHALF):
            if k + 1 < _NHALF:
                if k >= 1:
                    @pl.when(group_ok(k - 1))
                    def _():
                        scatter_wait(k - 1)
                idx_copy(k + 1)
                for d in g_descs(k + 1):
                    d.start()
            for d in g_descs(k):
                d.wait()

            @pl.when(group_ok(k))
            def _():
                scatter_issue(k)

        for k in (_NHALF - 2, _NHALF - 1):
            @pl.when(group_ok(k))
            def _():
                scatter_wait(k)

    return _sc_pool


def kernel(feat, rois):
    feat_t = jnp.transpose(feat[0], (1, 2, 0))
    padf = jnp.pad(feat_t, ((0, 8), (0, 8), (0, 0)), mode="edge")
    rois_t = jnp.transpose(rois)
    rois8 = jnp.concatenate(
        [rois_t, jnp.zeros((3, NROIS), jnp.float32)], axis=0)
    table = _build_table(padf)
    gidx = _build_idx(rois8)
    out2 = _make_sc_pool()(table, gidx)
    out6 = out2.reshape(BINS, NROIS // 8, 2, 8, C // 2)
    out = jnp.transpose(out6, (1, 3, 2, 4, 0))
    return out.reshape(NROIS, C, POOL, POOL)


# speedup vs baseline: 5.3851x; 1.2224x over previous
import functools

import jax
import jax.numpy as jnp
from jax import lax
from jax.experimental import pallas as pl
from jax.experimental.pallas import tpu as pltpu
from jax.experimental.pallas import tpu_sc as plsc

POOL = 7
SCALE = 0.0625
H = 50
W = 50
C = 256
NROIS = 1000
NSZ = 9
NTBL = NSZ * NSZ
ZROW = NTBL * H * W
RPAD = 1024
BINS = POOL * POOL
GIDX = 112
SIDX = 98
TROWS2 = (NTBL + 1) * H * W * 2


def _table_kernel(padf_ref, out_ref, a_ref, b_ref):
    t = pl.program_id(0)
    P = H + 8
    P2 = 2 * (W + 8)

    def emit_one(k):
        ww = k % NSZ

        @pl.when(k == 0)
        def _():
            a_ref[...] = padf_ref[...]

        @pl.when((ww == 0) & (k > 0))
        def _():
            a = a_ref[...]
            sh1 = jnp.concatenate([a[1:], a[P - 1:]], axis=0)
            a_ref[...] = jnp.maximum(a, sh1)

        b = b_ref[...]
        bs1 = jnp.concatenate([b[:, 2:, :], b[:, P2 - 2:, :]], axis=1)
        b_new = jnp.where(ww == 0, a_ref[0:H], jnp.maximum(b, bs1))
        b_ref[...] = b_new
        v = jnp.where(k >= NTBL, jnp.float32(0.0), b_new[:, 0:2 * W, :])
        return v.reshape(H * W * 2, C // 2)

    v0 = emit_one(2 * t)
    v1 = emit_one(2 * t + 1)
    out_ref[...] = jnp.concatenate([v0, v1], axis=0)


def _build_table(padf):
    return pl.pallas_call(
        _table_kernel,
        grid=((NTBL + 1) // 2,),
        in_specs=[pl.BlockSpec((H + 8, 2 * (W + 8), C // 2),
                               lambda t: (0, 0, 0))],
        out_specs=pl.BlockSpec((H * W * 4, C // 2), lambda t: (t, 0)),
        out_shape=jax.ShapeDtypeStruct((TROWS2, C // 2), jnp.float32),
        scratch_shapes=[
            pltpu.VMEM((H + 8, 2 * (W + 8), C // 2), jnp.float32),
            pltpu.VMEM((H, 2 * (W + 8), C // 2), jnp.float32),
        ],
    )(padf)


def _idx_kernel(rois_ref, gidx_ref):
    x1 = rois_ref[1:2, :]
    y1 = rois_ref[2:3, :]
    x2 = rois_ref[3:4, :]
    y2 = rois_ref[4:5, :]

    def bounds(lo, hi, size):
        start = jnp.round(lo * SCALE).astype(jnp.int32)
        end = jnp.round(hi * SCALE).astype(jnp.int32)
        length = jnp.maximum(end - start + 1, 1).astype(jnp.float32)
        binsz = length / float(POOL)
        p = lax.broadcasted_iota(jnp.int32, (POOL, 1), 0).astype(jnp.float32)
        bstart = jnp.clip(jnp.floor(p * binsz).astype(jnp.int32) + start, 0, size)
        bend = jnp.clip(jnp.ceil((p + 1.0) * binsz).astype(jnp.int32) + start, 0, size)
        return bstart, bend - bstart

    hs, szh = bounds(y1, y2, H)
    ws, szw = bounds(x1, x2, W)
    tbl = (szh[:, None, :] - 1) * NSZ + (szw[None, :, :] - 1)
    row = tbl * (H * W) + hs[:, None, :] * W + ws[None, :, :]
    empty = (szh[:, None, :] <= 0) | (szw[None, :, :] <= 0)
    q = jnp.where(empty, ZROW, row).reshape(BINS, NROIS)
    qp = jnp.concatenate(
        [q, jnp.full((BINS, RPAD - NROIS), ZROW, jnp.int32)], axis=1)
    q3 = qp.reshape(BINS, RPAD // 8, 8)
    ch = lax.broadcasted_iota(jnp.int32, (1, 1, 2, 1), 2)
    g4 = q3[:, :, None, :] * 2 + ch
    g4 = jnp.transpose(g4, (1, 0, 2, 3))
    half_a = g4[:, 0:25].reshape(RPAD // 8, 400)
    half_b = jnp.concatenate(
        [g4[:, 25:BINS].reshape(RPAD // 8, 384),
         jnp.full((RPAD // 8, 16), 2 * ZROW, jnp.int32)], axis=1)
    gidx_ref[...] = jnp.stack(
        [half_a, half_b], axis=1).reshape(RPAD // 4, 400)


def _build_idx(rois8):
    return pl.pallas_call(
        _idx_kernel,
        out_shape=jax.ShapeDtypeStruct((RPAD // 4, 400), jnp.int32),
    )(rois8)


_NC = 2
_NS = 16
_NW = _NC * _NS
_RPT = RPAD // _NW
_NHALF = 8
_GCH = ((0, 128), (128, 128), (256, 128), (384, 16))


@functools.cache
def _make_sc_pool():
    @functools.partial(
        pl.kernel,
        mesh=plsc.VectorSubcoreMesh(core_axis_name="c", subcore_axis_name="s"),
        compiler_params=pltpu.CompilerParams(needs_layout_passes=False),
        out_type=jax.ShapeDtypeStruct((2 * BINS * NROIS, C // 2), jnp.float32),
        scratch_types=[
            pltpu.VMEM((2, 400), jnp.int32),
            pltpu.VMEM((2, 400, C // 2), jnp.float32),
            pltpu.SemaphoreType.DMA((2,)),
            pltpu.SemaphoreType.DMA((2,)),
        ],
    )
    def _sc_pool(table_hbm, gidx_hbm, out_hbm, idx_v, rows, sem_g, sem_s):
        wid = lax.axis_index("s") * _NC + lax.axis_index("c")

        def idx_copy(k):
            pltpu.sync_copy(gidx_hbm.at[wid * _NHALF + k], idx_v.at[k & 1])

        def g_descs(k):
            b = k & 1
            return [
                pltpu.make_async_copy(
                    table_hbm.at[idx_v.at[b, pl.ds(o, n)]],
                    rows.at[b, pl.ds(o, n)], sem_g.at[b])
                for o, n in _GCH
            ]

        def group_ok(k):
            return (wid * 4 + k // 2) * 8 < NROIS

        def scatter_issue(k):
            b = k & 1
            h = k % 2
            nb = 25 - h
            rtg = wid * 4 + k // 2
            stride = 16 * (NROIS // 8)
            base = h * 25 * stride + rtg * 16

            def sc_body(e, c):
                dest = base + (e >> 1) * stride + (e & 1) * 8
                pltpu.async_copy(
                    rows.at[b, pl.ds(e * 8, 8)],
                    out_hbm.at[pl.ds(dest, 8)], sem_s.at[b])
                return c

            lax.fori_loop(0, 2 * nb, sc_body, 0)

        def scatter_wait(k):
            b = k & 1
            nb = 25 - (k % 2)

            def w_body(e, c):
                pltpu.make_async_copy(
                    rows.at[b, pl.ds(0, 8)],
                    out_hbm.at[pl.ds(0, 8)], sem_s.at[b]).wait()
                return c

            lax.fori_loop(0, 2 * nb, w_body, 0)

        idx_copy(0)
        for d in g_descs(0):
            d.start()
        for k in range(_NHALF):
            if k + 1 < _NHALF:
                if k >= 1:
                    @pl.when(group_ok(k - 1))
                    def _():
                        scatter_wait(k - 1)
                idx_copy(k + 1)
                for d in g_descs(k + 1):
                    d.start()
            for d in g_descs(k):
                d.wait()

            @pl.when(group_ok(k))
            def _():
                scatter_issue(k)

        for k in (_NHALF - 2, _NHALF - 1):
            @pl.when(group_ok(k))
            def _():
                scatter_wait(k)

    return _sc_pool


def kernel(feat, rois):
    feat_t = jnp.transpose(feat[0], (1, 2, 0))
    padf = jnp.pad(feat_t, ((0, 8), (0, 8), (0, 0)), mode="edge")
    padf = padf.reshape(H + 8, 2 * (W + 8), C // 2)
    rois_t = jnp.transpose(rois)
    rois8 = jnp.concatenate(
        [rois_t, jnp.zeros((3, NROIS), jnp.float32)], axis=0)
    table = _build_table(padf)
    gidx = _build_idx(rois8)
    out2 = _make_sc_pool()(table, gidx)
    out6 = out2.reshape(BINS, NROIS // 8, 2, 8, C // 2)
    out = jnp.transpose(out6, (1, 3, 2, 4, 0))
    return out.reshape(NROIS, C, POOL, POOL)
